# baseline (speedup 1.0000x reference)
import functools

import jax
import jax.numpy as jnp
from jax import lax
from jax.experimental import pallas as pl
from jax.experimental.pallas import tpu as pltpu
from jax.experimental.pallas import tpu_sc as plsc

_B, _N, _HID, _H, _NREL, _E = 64, 64, 256, 8, 64, 65536
_D = _HID // _H
_NN = _N * _N
_NP = 128

_NW = 32
_BPW = _B // _NW
_CH = 8192
_QCAP = 8192 + 16

_CDIM = (((1,), (1,)), ((), ()))


def _proj_body(x_ref, wq_ref, bq_ref, wk_ref, bk_ref, wv_ref, bv_ref, ee_ref,
               q_ref, k_ref, v_ref, qd_ref):
    x = x_ref[...]
    q = lax.dot_general(x, wq_ref[...], _CDIM, preferred_element_type=jnp.float32) + bq_ref[...]
    k = lax.dot_general(x, wk_ref[...], _CDIM, preferred_element_type=jnp.float32) + bk_ref[...]
    v = lax.dot_general(x, wv_ref[...], _CDIM, preferred_element_type=jnp.float32) + bv_ref[...]
    ee = ee_ref[...]
    nb = q.shape[0] // _N
    for h in range(_H):
        sl = slice(h * _D, (h + 1) * _D)
        for bb in range(nb):
            rl = slice(bb * _N, (bb + 1) * _N)
            q_ref[bb, h] = q[rl, sl]
            k_ref[bb, h] = k[rl, sl]
            v_ref[bb, h] = v[rl, sl]
        qd_ref[:, h * _NREL:(h + 1) * _NREL] = lax.dot_general(
            q[:, sl], ee[:, sl], _CDIM, preferred_element_type=jnp.float32)


def _proj(x, wq, bq, wk, bk, wv, bv, ee):
    rb = 512
    grid = (_B * _N // rb,)
    full = lambda shp: pl.BlockSpec(shp, lambda i: (0, 0))
    hd = pl.BlockSpec((rb // _N, _H, _N, _D), lambda i: (i, 0, 0, 0))
    return pl.pallas_call(
        _proj_body,
        grid=grid,
        in_specs=[
            pl.BlockSpec((rb, _HID), lambda i: (i, 0)),
            full((_HID, _HID)), full((1, _HID)),
            full((_HID, _HID)), full((1, _HID)),
            full((_HID, _HID)), full((1, _HID)),
            full((_NREL, _HID)),
        ],
        out_specs=[
            hd, hd, hd,
            pl.BlockSpec((rb, _H * _NREL), lambda i: (i, 0)),
        ],
        out_shape=[
            jax.ShapeDtypeStruct((_B, _H, _N, _D), jnp.float32),
            jax.ShapeDtypeStruct((_B, _H, _N, _D), jnp.float32),
            jax.ShapeDtypeStruct((_B, _H, _N, _D), jnp.float32),
            jax.ShapeDtypeStruct((_B * _N, _H * _NREL), jnp.float32),
        ],
    )(x, wq, bq, wk, bk, wv, bv, ee)


def _ekey_body(ei_ref, out_ref):
    eb = ei_ref[0]
    ei = ei_ref[1]
    ej = ei_ref[2]
    er = ei_ref[3]
    out_ref[...] = (eb << 18) | (ei << 12) | (ej << 6) | er


def _ekey(edge_indices):
    ei3 = edge_indices.reshape(4, 512, 128)
    out = pl.pallas_call(
        _ekey_body,
        out_shape=jax.ShapeDtypeStruct((512, 128), jnp.int32),
    )(ei3)
    return out.reshape(_E)


def _sc_body(ekey_hbm, qdot_hbm, corr_hbm, cnt_hbm,
             ek_v, q_v, qd_v, probe_v, comb_v):
    cid = lax.axis_index("c")
    sid = lax.axis_index("s")
    wid = cid * 16 + sid
    b0 = wid * _BPW
    iota = lax.iota(jnp.int32, 16)
    m9 = iota < 9
    h7 = iota & 7
    i9 = jnp.minimum(iota, 8)
    hq = h7 * _NREL
    is8 = iota == 8
    one = jnp.full((16,), 1, jnp.int32)
    onef = jnp.full((16,), 1.0, jnp.float32)
    zf = jnp.zeros((16,), jnp.float32)
    neg1 = jnp.full((16,), -1, jnp.int32)

    nav = jnp.zeros((16,), jnp.int32)
    with jax.named_scope("phase1"):
        for c in range(_E // _CH):
            pltpu.sync_copy(ekey_hbm.at[pl.ds(c * _CH, _CH)], ek_v)

            def scan_body(t, nav):
                for u in range(4):
                    key = ek_v[pl.ds(t * 64 + u * 16, 16)]
                    match = (key >> 19) == wid

                    @pl.when(jnp.any(match))
                    def _(nav=nav, key=key, match=match):
                        cs = plsc.cumsum(match.astype(jnp.int32))
                        plsc.store_scatter(q_v, [cs + (nav - one)], key, mask=match)

                    nav = nav + plsc.all_reduce_population_count(match)
                return nav

            nav = lax.fori_loop(0, _CH // 64, scan_body, nav)

    nq = nav[0]
    q_v[pl.ds(nq, 16)] = neg1

    for bi in range(_BPW):
        b = b0 + bi
        with jax.named_scope("qdotdma"):
            pltpu.sync_copy(qdot_hbm.at[pl.ds(b * _N, _N)], qd_v)
        with jax.named_scope("zero"):
            for hh in range(_H + 1):
                def zero_comb(z, _, hh=hh):
                    comb_v[hh, z, pl.ds(0, 16)] = zf
                    comb_v[hh, z, pl.ds(16, 16)] = zf
                    comb_v[hh, z, pl.ds(32, 16)] = zf
                    comb_v[hh, z, pl.ds(48, 16)] = zf
                    return 0
                lax.fori_loop(0, _N, zero_comb, 0)

        def group_body(g, _, b=b):
            key16 = q_v[pl.ds(g * 16, 16)]
            mb = (key16 >> 18) == b
            row = (key16 >> 12) & 63
            jrow = (key16 >> 6) & 63
            r16 = key16 & 63
            pidx = (key16 >> 6) & 4095
            plsc.store_scatter(probe_v, [pidx], iota, mask=mb)
            rbk = plsc.load_gather(probe_v, [pidx], mask=mb)
            okm = mb & (rbk == iota)
            coll = jnp.where(mb & (rbk != iota), one, 0)
            for h in range(_H):
                hv = jnp.full((16,), h, jnp.int32)
                vals = plsc.load_gather(qd_v, [row, r16 + h * _NREL], mask=okm)
                plsc.addupdate_scatter(comb_v, [hv, row, jrow], vals, mask=okm)
            plsc.addupdate_scatter(
                comb_v, [jnp.full((16,), _H, jnp.int32), row, jrow], onef, mask=okm)

            @pl.when(jnp.any(coll != 0))
            def _():
                for l in range(16):
                    keyb = jnp.full((16,), key16[l], jnp.int32)
                    msk = m9 & (jnp.full((16,), coll[l], jnp.int32) != 0)
                    rowb = (keyb >> 12) & 63
                    colb = (keyb & 63) + hq
                    jrowb = (keyb >> 6) & 63
                    vals = plsc.load_gather(qd_v, [rowb, colb], mask=msk)
                    vals = jnp.where(is8, jnp.float32(1.0), vals)
                    plsc.addupdate_scatter(comb_v, [i9, rowb, jrowb], vals, mask=msk)
            return 0

        with jax.named_scope("phase2"):
            lax.fori_loop(0, (nq + 15) // 16, group_body, 0)

        with jax.named_scope("outdma"):
            pltpu.sync_copy(comb_v.at[pl.ds(0, _H)], corr_hbm.at[b])
            pltpu.sync_copy(comb_v.at[pl.ds(_H, 1)], cnt_hbm.at[b])


def _sc_corr(ekey, qdot):
    mesh = plsc.VectorSubcoreMesh(core_axis_name="c", subcore_axis_name="s")
    f = pl.kernel(
        _sc_body,
        out_type=(
            jax.ShapeDtypeStruct((_B, _H, _N, _NP), jnp.float32),
            jax.ShapeDtypeStruct((_B, 1, _N, _NP), jnp.float32),
        ),
        mesh=mesh,
        compiler_params=pltpu.CompilerParams(needs_layout_passes=False),
        scratch_types=[
            pltpu.VMEM((_CH,), jnp.int32),
            pltpu.VMEM((_QCAP,), jnp.int32),
            pltpu.VMEM((_N, _H * _NREL), jnp.float32),
            pltpu.VMEM((_NN,), jnp.int32),
            pltpu.VMEM((_H + 1, _N, _NP), jnp.float32),
        ],
    )
    return f(ekey, qdot)


def _attn_body(q_ref, k_ref, v_ref, corr_ref, cnt_ref, out_ref):
    qh = q_ref[0]
    kh = k_ref[0]
    vh = v_ref[0]
    corr = corr_ref[0][:, :, :_N]
    mask = (cnt_ref[0, 0][:, :_N] > 0.0)[None, :, :]
    scale = jnp.float32(1.0) / jnp.sqrt(jnp.float32(_D))
    neg = jnp.float32(-jnp.inf)
    lg = lax.dot_general(qh, kh, (((2,), (2,)), ((0,), (0,))),
                         preferred_element_type=jnp.float32)
    lg = (lg - corr) * scale
    ml = jnp.where(mask, lg, neg)
    m = jnp.max(ml, axis=2, keepdims=True)
    m = jnp.where(jnp.isfinite(m), m, 0.0)
    e = jnp.exp(jnp.where(mask, lg - m, jnp.float32(-1e30)))
    s = jnp.sum(e, axis=2, keepdims=True)
    p = jnp.where(s > 0, e / jnp.where(s > 0, s, 1.0), 0.0)
    o = lax.dot_general(p, vh, (((2,), (1,)), ((0,), (0,))),
                        preferred_element_type=jnp.float32)
    for h in range(_H):
        out_ref[0, :, h * _D:(h + 1) * _D] = o[h]


def _attn(q, k, v, corr, cnt):
    hd = pl.BlockSpec((1, _H, _N, _D), lambda b: (b, 0, 0, 0))
    return pl.pallas_call(
        _attn_body,
        grid=(_B,),
        in_specs=[
            hd, hd, hd,
            pl.BlockSpec((1, _H, _N, _NP), lambda b: (b, 0, 0, 0)),
            pl.BlockSpec((1, 1, _N, _NP), lambda b: (b, 0, 0, 0)),
        ],
        out_specs=pl.BlockSpec((1, _N, _HID), lambda b: (b, 0, 0)),
        out_shape=jax.ShapeDtypeStruct((_B, _N, _HID), jnp.float32),
    )(q, k, v, corr, cnt)


def kernel(node_states, edge_indices, node_type_ids, Wq, bq, Wk, bk, Wv, bv, edge_emb):
    x = node_states.reshape(_B * _N, _HID)
    q, k, v, qdot = _proj(x, Wq, bq.reshape(1, _HID), Wk, bk.reshape(1, _HID),
                          Wv, bv.reshape(1, _HID), edge_emb)
    ekey = _ekey(edge_indices)
    corr, cnt = _sc_corr(ekey, qdot)
    out = _attn(q, k, v, corr, cnt)
    return out


# speedup vs baseline: 1.3123x; 1.3123x over previous
import functools

import jax
import jax.numpy as jnp
from jax import lax
from jax.experimental import pallas as pl
from jax.experimental.pallas import tpu as pltpu
from jax.experimental.pallas import tpu_sc as plsc

_B, _N, _HID, _H, _NREL, _E = 64, 64, 256, 8, 64, 65536
_D = _HID // _H
_NN = _N * _N
_NP = 128

_NW = 32
_BPW = _B // _NW
_CH = 8192
_QCAP = 8192 + 16

_CDIM = (((1,), (1,)), ((), ()))


def _proj_body(x_ref, wq_ref, bq_ref, wk_ref, bk_ref, wv_ref, bv_ref, ee_ref,
               q_ref, k_ref, v_ref, qd_ref):
    x = x_ref[...]
    q = lax.dot_general(x, wq_ref[...], _CDIM, preferred_element_type=jnp.float32) + bq_ref[...]
    k = lax.dot_general(x, wk_ref[...], _CDIM, preferred_element_type=jnp.float32) + bk_ref[...]
    v = lax.dot_general(x, wv_ref[...], _CDIM, preferred_element_type=jnp.float32) + bv_ref[...]
    ee = ee_ref[...]
    nb = q.shape[0] // _N
    for h in range(_H):
        sl = slice(h * _D, (h + 1) * _D)
        for bb in range(nb):
            rl = slice(bb * _N, (bb + 1) * _N)
            q_ref[bb, h] = q[rl, sl]
            k_ref[bb, h] = k[rl, sl]
            v_ref[bb, h] = v[rl, sl]
        qd_ref[:, h * _NREL:(h + 1) * _NREL] = lax.dot_general(
            q[:, sl], ee[:, sl], _CDIM, preferred_element_type=jnp.float32)


def _proj(x, wq, bq, wk, bk, wv, bv, ee):
    rb = 512
    grid = (_B * _N // rb,)
    full = lambda shp: pl.BlockSpec(shp, lambda i: (0, 0))
    hd = pl.BlockSpec((rb // _N, _H, _N, _D), lambda i: (i, 0, 0, 0))
    return pl.pallas_call(
        _proj_body,
        grid=grid,
        in_specs=[
            pl.BlockSpec((rb, _HID), lambda i: (i, 0)),
            full((_HID, _HID)), full((1, _HID)),
            full((_HID, _HID)), full((1, _HID)),
            full((_HID, _HID)), full((1, _HID)),
            full((_NREL, _HID)),
        ],
        out_specs=[
            hd, hd, hd,
            pl.BlockSpec((rb, _H * _NREL), lambda i: (i, 0)),
        ],
        out_shape=[
            jax.ShapeDtypeStruct((_B, _H, _N, _D), jnp.float32),
            jax.ShapeDtypeStruct((_B, _H, _N, _D), jnp.float32),
            jax.ShapeDtypeStruct((_B, _H, _N, _D), jnp.float32),
            jax.ShapeDtypeStruct((_B * _N, _H * _NREL), jnp.float32),
        ],
    )(x, wq, bq, wk, bk, wv, bv, ee)


def _ekey_body(ei_ref, out_ref):
    eb = ei_ref[0]
    ei = ei_ref[1]
    ej = ei_ref[2]
    er = ei_ref[3]
    out_ref[...] = (eb << 18) | (ei << 12) | (ej << 6) | er


def _ekey(edge_indices):
    ei3 = edge_indices.reshape(4, 512, 128)
    out = pl.pallas_call(
        _ekey_body,
        out_shape=jax.ShapeDtypeStruct((512, 128), jnp.int32),
    )(ei3)
    return out.reshape(_E)


def _sc_body(ekey_hbm, qdot_hbm, corr_hbm, cnt_hbm,
             ek_v, q_v, qd_v, probe_v, comb_v):
    cid = lax.axis_index("c")
    sid = lax.axis_index("s")
    wid = cid * 16 + sid
    b0 = wid * _BPW
    iota = lax.iota(jnp.int32, 16)
    m9 = iota < 9
    h7 = iota & 7
    i9 = jnp.minimum(iota, 8)
    hq = h7 * _NREL
    is8 = iota == 8
    one = jnp.full((16,), 1, jnp.int32)
    onef = jnp.full((16,), 1.0, jnp.float32)
    zf = jnp.zeros((16,), jnp.float32)
    neg1 = jnp.full((16,), -1, jnp.int32)

    nav = jnp.zeros((16,), jnp.int32)
    with jax.named_scope("phase1"):
        for c in range(_E // _CH):
            pltpu.sync_copy(ekey_hbm.at[pl.ds(c * _CH, _CH)], ek_v)

            def scan_body(t, nav):
                for u in range(4):
                    key = ek_v[pl.ds(t * 64 + u * 16, 16)]
                    match = (key >> 19) == wid
                    cs = plsc.cumsum(match.astype(jnp.int32))
                    plsc.store_scatter(q_v, [cs + (nav - one)], key, mask=match)
                    nav = nav + plsc.all_reduce_population_count(match)
                return nav

            nav = lax.fori_loop(0, _CH // 64, scan_body, nav)

    nq = nav[0]
    q_v[pl.ds(nq, 16)] = neg1

    for bi in range(_BPW):
        b = b0 + bi
        with jax.named_scope("qdotdma"):
            pltpu.sync_copy(qdot_hbm.at[pl.ds(b * _N, _N)], qd_v)
        with jax.named_scope("zero"):
            for hh in range(_H + 1):
                def zero_comb(z, _, hh=hh):
                    comb_v[hh, z, pl.ds(0, 16)] = zf
                    comb_v[hh, z, pl.ds(16, 16)] = zf
                    comb_v[hh, z, pl.ds(32, 16)] = zf
                    comb_v[hh, z, pl.ds(48, 16)] = zf
                    return 0
                lax.fori_loop(0, _N, zero_comb, 0)

        def group_body(g, _, b=b):
            key16 = q_v[pl.ds(g * 16, 16)]
            mb = (key16 >> 18) == b
            row = (key16 >> 12) & 63
            jrow = (key16 >> 6) & 63
            r16 = key16 & 63
            pidx = (key16 >> 6) & 4095
            plsc.store_scatter(probe_v, [pidx], iota, mask=mb)
            rbk = plsc.load_gather(probe_v, [pidx], mask=mb)
            okm = mb & (rbk == iota)
            coll = jnp.where(mb & (rbk != iota), one, 0)
            for h in range(_H):
                hv = jnp.full((16,), h, jnp.int32)
                vals = plsc.load_gather(qd_v, [row, r16 + h * _NREL], mask=okm)
                plsc.addupdate_scatter(comb_v, [hv, row, jrow], vals, mask=okm)
            plsc.addupdate_scatter(
                comb_v, [jnp.full((16,), _H, jnp.int32), row, jrow], onef, mask=okm)

            @pl.when(jnp.any(coll != 0))
            def _():
                for l in range(16):
                    keyb = jnp.full((16,), key16[l], jnp.int32)
                    msk = m9 & (jnp.full((16,), coll[l], jnp.int32) != 0)
                    rowb = (keyb >> 12) & 63
                    colb = (keyb & 63) + hq
                    jrowb = (keyb >> 6) & 63
                    vals = plsc.load_gather(qd_v, [rowb, colb], mask=msk)
                    vals = jnp.where(is8, jnp.float32(1.0), vals)
                    plsc.addupdate_scatter(comb_v, [i9, rowb, jrowb], vals, mask=msk)
            return 0

        with jax.named_scope("phase2"):
            lax.fori_loop(0, (nq + 15) // 16, group_body, 0)

        with jax.named_scope("outdma"):
            pltpu.sync_copy(comb_v.at[pl.ds(0, _H)], corr_hbm.at[b])
            pltpu.sync_copy(comb_v.at[pl.ds(_H, 1)], cnt_hbm.at[b])


def _sc_corr(ekey, qdot):
    mesh = plsc.VectorSubcoreMesh(core_axis_name="c", subcore_axis_name="s")
    f = pl.kernel(
        _sc_body,
        out_type=(
            jax.ShapeDtypeStruct((_B, _H, _N, _NP), jnp.float32),
            jax.ShapeDtypeStruct((_B, 1, _N, _NP), jnp.float32),
        ),
        mesh=mesh,
        compiler_params=pltpu.CompilerParams(needs_layout_passes=False),
        scratch_types=[
            pltpu.VMEM((_CH,), jnp.int32),
            pltpu.VMEM((_QCAP,), jnp.int32),
            pltpu.VMEM((_N, _H * _NREL), jnp.float32),
            pltpu.VMEM((_NN,), jnp.int32),
            pltpu.VMEM((_H + 1, _N, _NP), jnp.float32),
        ],
    )
    return f(ekey, qdot)


def _attn_body(q_ref, k_ref, v_ref, corr_ref, cnt_ref, out_ref):
    qh = q_ref[0]
    kh = k_ref[0]
    vh = v_ref[0]
    corr = corr_ref[0][:, :, :_N]
    mask = (cnt_ref[0, 0][:, :_N] > 0.0)[None, :, :]
    scale = jnp.float32(1.0) / jnp.sqrt(jnp.float32(_D))
    neg = jnp.float32(-jnp.inf)
    lg = lax.dot_general(qh, kh, (((2,), (2,)), ((0,), (0,))),
                         preferred_element_type=jnp.float32)
    lg = (lg - corr) * scale
    ml = jnp.where(mask, lg, neg)
    m = jnp.max(ml, axis=2, keepdims=True)
    m = jnp.where(jnp.isfinite(m), m, 0.0)
    e = jnp.exp(jnp.where(mask, lg - m, jnp.float32(-1e30)))
    s = jnp.sum(e, axis=2, keepdims=True)
    p = jnp.where(s > 0, e / jnp.where(s > 0, s, 1.0), 0.0)
    o = lax.dot_general(p, vh, (((2,), (1,)), ((0,), (0,))),
                        preferred_element_type=jnp.float32)
    for h in range(_H):
        out_ref[0, :, h * _D:(h + 1) * _D] = o[h]


def _attn(q, k, v, corr, cnt):
    hd = pl.BlockSpec((1, _H, _N, _D), lambda b: (b, 0, 0, 0))
    return pl.pallas_call(
        _attn_body,
        grid=(_B,),
        in_specs=[
            hd, hd, hd,
            pl.BlockSpec((1, _H, _N, _NP), lambda b: (b, 0, 0, 0)),
            pl.BlockSpec((1, 1, _N, _NP), lambda b: (b, 0, 0, 0)),
        ],
        out_specs=pl.BlockSpec((1, _N, _HID), lambda b: (b, 0, 0)),
        out_shape=jax.ShapeDtypeStruct((_B, _N, _HID), jnp.float32),
    )(q, k, v, corr, cnt)


def kernel(node_states, edge_indices, node_type_ids, Wq, bq, Wk, bk, Wv, bv, edge_emb):
    x = node_states.reshape(_B * _N, _HID)
    q, k, v, qdot = _proj(x, Wq, bq.reshape(1, _HID), Wk, bk.reshape(1, _HID),
                          Wv, bv.reshape(1, _HID), edge_emb)
    ekey = _ekey(edge_indices)
    corr, cnt = _sc_corr(ekey, qdot)
    out = _attn(q, k, v, corr, cnt)
    return out


# speedup vs baseline: 1.3792x; 1.0510x over previous
import functools

import jax
import jax.numpy as jnp
from jax import lax
from jax.experimental import pallas as pl
from jax.experimental.pallas import tpu as pltpu
from jax.experimental.pallas import tpu_sc as plsc

_B, _N, _HID, _H, _NREL, _E = 64, 64, 256, 8, 64, 65536
_D = _HID // _H
_NN = _N * _N
_NP = 128

_NW = 32
_BPW = _B // _NW
_CH = 8192
_QCAP = 8192 + 16

_ABB = 2
_CDIM = (((1,), (1,)), ((), ()))


def _proj_body(x_ref, wq_ref, bq_ref, wk_ref, bk_ref, wv_ref, bv_ref, ee_ref,
               q_ref, k_ref, v_ref, qd_ref):
    x = x_ref[...]
    q = lax.dot_general(x, wq_ref[...], _CDIM, preferred_element_type=jnp.float32) + bq_ref[...]
    k = lax.dot_general(x, wk_ref[...], _CDIM, preferred_element_type=jnp.float32) + bk_ref[...]
    v = lax.dot_general(x, wv_ref[...], _CDIM, preferred_element_type=jnp.float32) + bv_ref[...]
    ee = ee_ref[...]
    nb = q.shape[0] // _N
    for h in range(_H):
        sl = slice(h * _D, (h + 1) * _D)
        kt = k[:, sl].T
        for bb in range(nb):
            rl = slice(bb * _N, (bb + 1) * _N)
            q_ref[bb, h] = q[rl, sl]
            k_ref[bb, h] = kt[:, rl]
            v_ref[bb, h] = v[rl, sl]
        qd_ref[:, h * _NREL:(h + 1) * _NREL] = lax.dot_general(
            q[:, sl], ee[:, sl], _CDIM, preferred_element_type=jnp.float32)


def _proj(x, wq, bq, wk, bk, wv, bv, ee):
    rb = 512
    grid = (_B * _N // rb,)
    full = lambda shp: pl.BlockSpec(shp, lambda i: (0, 0))
    hd = pl.BlockSpec((rb // _N, _H, _N, _D), lambda i: (i, 0, 0, 0))
    hdt = pl.BlockSpec((rb // _N, _H, _D, _N), lambda i: (i, 0, 0, 0))
    return pl.pallas_call(
        _proj_body,
        grid=grid,
        in_specs=[
            pl.BlockSpec((rb, _HID), lambda i: (i, 0)),
            full((_HID, _HID)), full((1, _HID)),
            full((_HID, _HID)), full((1, _HID)),
            full((_HID, _HID)), full((1, _HID)),
            full((_NREL, _HID)),
        ],
        out_specs=[
            hd, hdt, hd,
            pl.BlockSpec((rb, _H * _NREL), lambda i: (i, 0)),
        ],
        out_shape=[
            jax.ShapeDtypeStruct((_B, _H, _N, _D), jnp.float32),
            jax.ShapeDtypeStruct((_B, _H, _D, _N), jnp.float32),
            jax.ShapeDtypeStruct((_B, _H, _N, _D), jnp.float32),
            jax.ShapeDtypeStruct((_B * _N, _H * _NREL), jnp.float32),
        ],
    )(x, wq, bq, wk, bk, wv, bv, ee)


def _ekey_body(ei_ref, out_ref):
    eb = ei_ref[0]
    ei = ei_ref[1]
    ej = ei_ref[2]
    er = ei_ref[3]
    out_ref[...] = (eb << 18) | (ei << 12) | (ej << 6) | er


def _ekey(edge_indices):
    ei3 = edge_indices.reshape(4, 512, 128)
    out = pl.pallas_call(
        _ekey_body,
        out_shape=jax.ShapeDtypeStruct((512, 128), jnp.int32),
    )(ei3)
    return out.reshape(_E)


def _sc_body(ekey_hbm, qdot_hbm, corr_hbm, cnt_hbm,
             ek_v, q_v, qd_v, probe_v, comb_v):
    cid = lax.axis_index("c")
    sid = lax.axis_index("s")
    wid = cid * 16 + sid
    b0 = wid * _BPW
    iota = lax.iota(jnp.int32, 16)
    m9 = iota < 9
    h7 = iota & 7
    i9 = jnp.minimum(iota, 8)
    hq = h7 * _NREL
    is8 = iota == 8
    one = jnp.full((16,), 1, jnp.int32)
    onef = jnp.full((16,), 1.0, jnp.float32)
    zf = jnp.zeros((16,), jnp.float32)
    neg1 = jnp.full((16,), -1, jnp.int32)

    nav = jnp.zeros((16,), jnp.int32)
    with jax.named_scope("phase1"):
        for c in range(_E // _CH):
            pltpu.sync_copy(ekey_hbm.at[pl.ds(c * _CH, _CH)], ek_v)

            def scan_body(t, nav):
                for u in range(4):
                    key = ek_v[pl.ds(t * 64 + u * 16, 16)]
                    match = (key >> 19) == wid
                    cs = plsc.cumsum(match.astype(jnp.int32))
                    plsc.store_scatter(q_v, [cs + (nav - one)], key, mask=match)
                    nav = nav + plsc.all_reduce_population_count(match)
                return nav

            nav = lax.fori_loop(0, _CH // 64, scan_body, nav)

    nq = nav[0]
    q_v[pl.ds(nq, 16)] = neg1

    for bi in range(_BPW):
        b = b0 + bi
        with jax.named_scope("qdotdma"):
            pltpu.sync_copy(qdot_hbm.at[pl.ds(b * _N, _N)], qd_v)
        with jax.named_scope("zero"):
            for hh in range(_H + 1):
                def zero_comb(z, _, hh=hh):
                    comb_v[hh, z, pl.ds(0, 16)] = zf
                    comb_v[hh, z, pl.ds(16, 16)] = zf
                    comb_v[hh, z, pl.ds(32, 16)] = zf
                    comb_v[hh, z, pl.ds(48, 16)] = zf
                    return 0
                lax.fori_loop(0, _N, zero_comb, 0)

        def group_body(g, _, b=b):
            key16 = q_v[pl.ds(g * 16, 16)]
            mb = (key16 >> 18) == b
            row = (key16 >> 12) & 63
            jrow = (key16 >> 6) & 63
            r16 = key16 & 63
            pidx = (key16 >> 6) & 4095
            plsc.store_scatter(probe_v, [pidx], iota, mask=mb)
            rbk = plsc.load_gather(probe_v, [pidx], mask=mb)
            okm = mb & (rbk == iota)
            coll = jnp.where(mb & (rbk != iota), one, 0)
            for h in range(_H):
                hv = jnp.full((16,), h, jnp.int32)
                vals = plsc.load_gather(qd_v, [row, r16 + h * _NREL], mask=okm)
                plsc.addupdate_scatter(comb_v, [hv, row, jrow], vals, mask=okm)
            plsc.addupdate_scatter(
                comb_v, [jnp.full((16,), _H, jnp.int32), row, jrow], onef, mask=okm)

            @pl.when(jnp.any(coll != 0))
            def _():
                for l in range(16):
                    keyb = jnp.full((16,), key16[l], jnp.int32)
                    msk = m9 & (jnp.full((16,), coll[l], jnp.int32) != 0)
                    rowb = (keyb >> 12) & 63
                    colb = (keyb & 63) + hq
                    jrowb = (keyb >> 6) & 63
                    vals = plsc.load_gather(qd_v, [rowb, colb], mask=msk)
                    vals = jnp.where(is8, jnp.float32(1.0), vals)
                    plsc.addupdate_scatter(comb_v, [i9, rowb, jrowb], vals, mask=msk)
            return 0

        with jax.named_scope("phase2"):
            lax.fori_loop(0, (nq + 15) // 16, group_body, 0)

        with jax.named_scope("outdma"):
            pltpu.sync_copy(comb_v.at[pl.ds(0, _H)], corr_hbm.at[b])
            pltpu.sync_copy(comb_v.at[pl.ds(_H, 1)], cnt_hbm.at[b])


def _sc_corr(ekey, qdot):
    mesh = plsc.VectorSubcoreMesh(core_axis_name="c", subcore_axis_name="s")
    f = pl.kernel(
        _sc_body,
        out_type=(
            jax.ShapeDtypeStruct((_B, _H, _N, _NP), jnp.float32),
            jax.ShapeDtypeStruct((_B, 1, _N, _NP), jnp.float32),
        ),
        mesh=mesh,
        compiler_params=pltpu.CompilerParams(needs_layout_passes=False),
        scratch_types=[
            pltpu.VMEM((_CH,), jnp.int32),
            pltpu.VMEM((_QCAP,), jnp.int32),
            pltpu.VMEM((_N, _H * _NREL), jnp.float32),
            pltpu.VMEM((_NN,), jnp.int32),
            pltpu.VMEM((_H + 1, _N, _NP), jnp.float32),
        ],
    )
    return f(ekey, qdot)


def _attn_body(q_ref, k_ref, v_ref, corr_ref, cnt_ref, out_ref):
    scale = jnp.float32(1.0) / jnp.sqrt(jnp.float32(_D))
    for bb in range(_ABB):
        qh = q_ref[bb]
        kt = k_ref[bb]
        vh = v_ref[bb]
        corr = corr_ref[bb][:, :, :_N]
        maskb = (cnt_ref[bb, 0][:, :_N] > 0.0)[None, :, :]
        maskf = maskb.astype(jnp.float32)
        lg = lax.dot_general(qh, kt, (((2,), (1,)), ((0,), (0,))),
                             preferred_element_type=jnp.float32)
        lg = (lg - corr) * scale
        lgm = jnp.where(maskb, lg, jnp.float32(-1e30))
        m = jnp.max(lgm, axis=2, keepdims=True)
        e = jnp.exp(lgm - m) * maskf
        s = jnp.sum(e, axis=2, keepdims=True)
        p = e / jnp.where(s > 0, s, jnp.float32(1.0))
        o = lax.dot_general(p, vh, (((2,), (1,)), ((0,), (0,))),
                            preferred_element_type=jnp.float32)
        for h in range(_H):
            out_ref[bb, :, h * _D:(h + 1) * _D] = o[h]


def _attn(q, k, v, corr, cnt):
    return pl.pallas_call(
        _attn_body,
        grid=(_B // _ABB,),
        in_specs=[
            pl.BlockSpec((_ABB, _H, _N, _D), lambda b: (b, 0, 0, 0)),
            pl.BlockSpec((_ABB, _H, _D, _N), lambda b: (b, 0, 0, 0)),
            pl.BlockSpec((_ABB, _H, _N, _D), lambda b: (b, 0, 0, 0)),
            pl.BlockSpec((_ABB, _H, _N, _NP), lambda b: (b, 0, 0, 0)),
            pl.BlockSpec((_ABB, 1, _N, _NP), lambda b: (b, 0, 0, 0)),
        ],
        out_specs=pl.BlockSpec((_ABB, _N, _HID), lambda b: (b, 0, 0)),
        out_shape=jax.ShapeDtypeStruct((_B, _N, _HID), jnp.float32),
    )(q, k, v, corr, cnt)


def kernel(node_states, edge_indices, node_type_ids, Wq, bq, Wk, bk, Wv, bv, edge_emb):
    x = node_states.reshape(_B * _N, _HID)
    q, k, v, qdot = _proj(x, Wq, bq.reshape(1, _HID), Wk, bk.reshape(1, _HID),
                          Wv, bv.reshape(1, _HID), edge_emb)
    ekey = _ekey(edge_indices)
    corr, cnt = _sc_corr(ekey, qdot)
    out = _attn(q, k, v, corr, cnt)
    return out


# speedup vs baseline: 1.3989x; 1.0143x over previous
import functools

import jax
import jax.numpy as jnp
from jax import lax
from jax.experimental import pallas as pl
from jax.experimental.pallas import tpu as pltpu
from jax.experimental.pallas import tpu_sc as plsc

_B, _N, _HID, _H, _NREL, _E = 64, 64, 256, 8, 64, 65536
_D = _HID // _H
_NN = _N * _N
_NP = 128

_NW = 32
_BPW = _B // _NW
_CH = 8192
_QCAP = 8192 + 16

_ABB = 2
_CDIM = (((1,), (1,)), ((), ()))


def _proj_body(x_ref, wq_ref, bq_ref, wk_ref, bk_ref, wv_ref, bv_ref, ee_ref,
               q_ref, k_ref, v_ref, qd_ref):
    x = x_ref[...]
    q = lax.dot_general(x, wq_ref[...], _CDIM, preferred_element_type=jnp.float32) + bq_ref[...]
    k = lax.dot_general(x, wk_ref[...], _CDIM, preferred_element_type=jnp.float32) + bk_ref[...]
    v = lax.dot_general(x, wv_ref[...], _CDIM, preferred_element_type=jnp.float32) + bv_ref[...]
    ee = ee_ref[...]
    nb = q.shape[0] // _N
    for h in range(_H):
        sl = slice(h * _D, (h + 1) * _D)
        kt = k[:, sl].T
        for bb in range(nb):
            rl = slice(bb * _N, (bb + 1) * _N)
            q_ref[bb, h] = q[rl, sl]
            k_ref[bb, h] = kt[:, rl]
            v_ref[bb, h] = v[rl, sl]
        qd_ref[:, h * _NREL:(h + 1) * _NREL] = lax.dot_general(
            q[:, sl], ee[:, sl], _CDIM, preferred_element_type=jnp.float32)


def _proj(x, wq, bq, wk, bk, wv, bv, ee):
    rb = 512
    grid = (_B * _N // rb,)
    full = lambda shp: pl.BlockSpec(shp, lambda i: (0, 0))
    hd = pl.BlockSpec((rb // _N, _H, _N, _D), lambda i: (i, 0, 0, 0))
    hdt = pl.BlockSpec((rb // _N, _H, _D, _N), lambda i: (i, 0, 0, 0))
    return pl.pallas_call(
        _proj_body,
        grid=grid,
        in_specs=[
            pl.BlockSpec((rb, _HID), lambda i: (i, 0)),
            full((_HID, _HID)), full((1, _HID)),
            full((_HID, _HID)), full((1, _HID)),
            full((_HID, _HID)), full((1, _HID)),
            full((_NREL, _HID)),
        ],
        out_specs=[
            hd, hdt, hd,
            pl.BlockSpec((rb, _H * _NREL), lambda i: (i, 0)),
        ],
        out_shape=[
            jax.ShapeDtypeStruct((_B, _H, _N, _D), jnp.float32),
            jax.ShapeDtypeStruct((_B, _H, _D, _N), jnp.float32),
            jax.ShapeDtypeStruct((_B, _H, _N, _D), jnp.float32),
            jax.ShapeDtypeStruct((_B * _N, _H * _NREL), jnp.float32),
        ],
    )(x, wq, bq, wk, bk, wv, bv, ee)


def _ekey_body(ei_ref, out_ref):
    eb = ei_ref[0]
    ei = ei_ref[1]
    ej = ei_ref[2]
    er = ei_ref[3]
    out_ref[...] = (eb << 18) | (ei << 12) | (ej << 6) | er


def _ekey(edge_indices):
    ei3 = edge_indices.reshape(4, 512, 128)
    out = pl.pallas_call(
        _ekey_body,
        out_shape=jax.ShapeDtypeStruct((512, 128), jnp.int32),
    )(ei3)
    return out.reshape(_E)


def _sc_body(ekey_hbm, qdot_hbm, corr_hbm, cnt_hbm,
             ek_v, q_v, qd_v, probe_v, comb_v):
    cid = lax.axis_index("c")
    sid = lax.axis_index("s")
    wid = cid * 16 + sid
    b0 = wid * _BPW
    iota = lax.iota(jnp.int32, 16)
    m9 = iota < 9
    h7 = iota & 7
    i9 = jnp.minimum(iota, 8)
    hq = h7 * _NREL
    is8 = iota == 8
    one = jnp.full((16,), 1, jnp.int32)
    onef = jnp.full((16,), 1.0, jnp.float32)
    zf = jnp.zeros((16,), jnp.float32)
    neg1 = jnp.full((16,), -1, jnp.int32)

    nav = jnp.full((16,), -1, jnp.int32)
    with jax.named_scope("phase1"):
        for c in range(_E // _CH):
            pltpu.sync_copy(ekey_hbm.at[pl.ds(c * _CH, _CH)], ek_v)

            def scan_body(t, navm1):
                for u in range(8):
                    key = ek_v[pl.ds(t * 128 + u * 16, 16)]
                    match = (key >> 19) == wid
                    cs = plsc.cumsum(match.astype(jnp.int32))
                    plsc.store_scatter(q_v, [cs + navm1], key, mask=match)
                    navm1 = navm1 + plsc.all_reduce_population_count(match)
                return navm1

            nav = lax.fori_loop(0, _CH // 128, scan_body, nav)

    nq = nav[0] + 1
    q_v[pl.ds(nq, 16)] = neg1

    for bi in range(_BPW):
        b = b0 + bi
        with jax.named_scope("qdotdma"):
            pltpu.sync_copy(qdot_hbm.at[pl.ds(b * _N, _N)], qd_v)
        with jax.named_scope("zero"):
            for hh in range(_H + 1):
                def zero_comb(z, _, hh=hh):
                    comb_v[hh, z, pl.ds(0, 16)] = zf
                    comb_v[hh, z, pl.ds(16, 16)] = zf
                    comb_v[hh, z, pl.ds(32, 16)] = zf
                    comb_v[hh, z, pl.ds(48, 16)] = zf
                    return 0
                lax.fori_loop(0, _N, zero_comb, 0)

        def group_body(g, _, b=b):
            key16 = q_v[pl.ds(g * 16, 16)]
            mb = (key16 >> 18) == b
            row = (key16 >> 12) & 63
            jrow = (key16 >> 6) & 63
            r16 = key16 & 63
            pidx = (key16 >> 6) & 4095
            plsc.store_scatter(probe_v, [pidx], iota, mask=mb)
            rbk = plsc.load_gather(probe_v, [pidx], mask=mb)
            okm = mb & (rbk == iota)
            coll = jnp.where(mb & (rbk != iota), one, 0)
            for h in range(_H):
                hv = jnp.full((16,), h, jnp.int32)
                vals = plsc.load_gather(qd_v, [row, r16 + h * _NREL], mask=okm)
                plsc.addupdate_scatter(comb_v, [hv, row, jrow], vals, mask=okm)
            plsc.addupdate_scatter(
                comb_v, [jnp.full((16,), _H, jnp.int32), row, jrow], onef, mask=okm)

            @pl.when(jnp.any(coll != 0))
            def _():
                for l in range(16):
                    keyb = jnp.full((16,), key16[l], jnp.int32)
                    msk = m9 & (jnp.full((16,), coll[l], jnp.int32) != 0)
                    rowb = (keyb >> 12) & 63
                    colb = (keyb & 63) + hq
                    jrowb = (keyb >> 6) & 63
                    vals = plsc.load_gather(qd_v, [rowb, colb], mask=msk)
                    vals = jnp.where(is8, jnp.float32(1.0), vals)
                    plsc.addupdate_scatter(comb_v, [i9, rowb, jrowb], vals, mask=msk)
            return 0

        with jax.named_scope("phase2"):
            lax.fori_loop(0, (nq + 15) // 16, group_body, 0)

        with jax.named_scope("outdma"):
            pltpu.sync_copy(comb_v.at[pl.ds(0, _H)], corr_hbm.at[b])
            pltpu.sync_copy(comb_v.at[pl.ds(_H, 1)], cnt_hbm.at[b])


def _sc_corr(ekey, qdot):
    mesh = plsc.VectorSubcoreMesh(core_axis_name="c", subcore_axis_name="s")
    f = pl.kernel(
        _sc_body,
        out_type=(
            jax.ShapeDtypeStruct((_B, _H, _N, _NP), jnp.float32),
            jax.ShapeDtypeStruct((_B, 1, _N, _NP), jnp.float32),
        ),
        mesh=mesh,
        compiler_params=pltpu.CompilerParams(needs_layout_passes=False),
        scratch_types=[
            pltpu.VMEM((_CH,), jnp.int32),
            pltpu.VMEM((_QCAP,), jnp.int32),
            pltpu.VMEM((_N, _H * _NREL), jnp.float32),
            pltpu.VMEM((_NN,), jnp.int32),
            pltpu.VMEM((_H + 1, _N, _NP), jnp.float32),
        ],
    )
    return f(ekey, qdot)


def _attn_body(q_ref, k_ref, v_ref, corr_ref, cnt_ref, out_ref):
    scale = jnp.float32(1.0) / jnp.sqrt(jnp.float32(_D))
    for bb in range(_ABB):
        qh = q_ref[bb]
        kt = k_ref[bb]
        vh = v_ref[bb]
        corr = corr_ref[bb][:, :, :_N]
        maskb = (cnt_ref[bb, 0][:, :_N] > 0.0)[None, :, :]
        maskf = maskb.astype(jnp.float32)
        lg = lax.dot_general(qh, kt, (((2,), (1,)), ((0,), (0,))),
                             preferred_element_type=jnp.float32)
        lg = (lg - corr) * scale
        lgm = jnp.where(maskb, lg, jnp.float32(-1e30))
        m = jnp.max(lgm, axis=2, keepdims=True)
        e = jnp.exp(lgm - m) * maskf
        s = jnp.sum(e, axis=2, keepdims=True)
        p = e / jnp.where(s > 0, s, jnp.float32(1.0))
        o = lax.dot_general(p, vh, (((2,), (1,)), ((0,), (0,))),
                            preferred_element_type=jnp.float32)
        for h in range(_H):
            out_ref[bb, :, h * _D:(h + 1) * _D] = o[h]


def _attn(q, k, v, corr, cnt):
    return pl.pallas_call(
        _attn_body,
        grid=(_B // _ABB,),
        in_specs=[
            pl.BlockSpec((_ABB, _H, _N, _D), lambda b: (b, 0, 0, 0)),
            pl.BlockSpec((_ABB, _H, _D, _N), lambda b: (b, 0, 0, 0)),
            pl.BlockSpec((_ABB, _H, _N, _D), lambda b: (b, 0, 0, 0)),
            pl.BlockSpec((_ABB, _H, _N, _NP), lambda b: (b, 0, 0, 0)),
            pl.BlockSpec((_ABB, 1, _N, _NP), lambda b: (b, 0, 0, 0)),
        ],
        out_specs=pl.BlockSpec((_ABB, _N, _HID), lambda b: (b, 0, 0)),
        out_shape=jax.ShapeDtypeStruct((_B, _N, _HID), jnp.float32),
    )(q, k, v, corr, cnt)


def kernel(node_states, edge_indices, node_type_ids, Wq, bq, Wk, bk, Wv, bv, edge_emb):
    x = node_states.reshape(_B * _N, _HID)
    q, k, v, qdot = _proj(x, Wq, bq.reshape(1, _HID), Wk, bk.reshape(1, _HID),
                          Wv, bv.reshape(1, _HID), edge_emb)
    ekey = _ekey(edge_indices)
    corr, cnt = _sc_corr(ekey, qdot)
    out = _attn(q, k, v, corr, cnt)
    return out
